# Initial kernel scaffold; baseline (speedup 1.0000x reference)
import jax
import jax.numpy as jnp
from jax.experimental import pallas as pl


def kernel(positions, senders, receivers):
    raise NotImplementedError("write your pallas kernel here")


# speedup vs baseline: 12.5829x; 12.5829x over previous
import functools

import jax
import jax.numpy as jnp
from jax import lax
from jax.experimental import pallas as pl
from jax.experimental.pallas import tpu as pltpu
from jax.experimental.pallas import tpu_sc as plsc

N_NODES = 100000
N_EDGES = 6400000

NC = 2
NS = 16
L = 16
NW = NC * NS

SUB = 128
NSUB = 16
CHUNK = SUB * NSUB
N_CHUNKS = N_EDGES // CHUNK

_MESH = plsc.VectorSubcoreMesh(core_axis_name="c", subcore_axis_name="s")


def _rsqrt(x):
    i = lax.bitcast_convert_type(x, jnp.int32)
    i = jnp.int32(0x5F3759DF) - lax.shift_right_logical(i, 1)
    y = lax.bitcast_convert_type(i, jnp.float32)
    for _ in range(3):
        y = y * (1.5 - 0.5 * x * y * y)
    return y


@functools.partial(
    pl.kernel,
    out_type=jax.ShapeDtypeStruct((N_EDGES, 4), jnp.float32),
    mesh=_MESH,
    scratch_types=[
        pltpu.VMEM((NSUB, SUB), jnp.int32),
        pltpu.VMEM((NSUB, SUB), jnp.int32),
        pltpu.VMEM((CHUNK, 3), jnp.float32),
        pltpu.VMEM((CHUNK, 3), jnp.float32),
        pltpu.VMEM((CHUNK, 4), jnp.float32),
        pltpu.SemaphoreType.DMA,
    ],
    compiler_params=pltpu.CompilerParams(
        needs_layout_passes=False, use_tc_tiling_on_sc=False),
)
def _edge_vectors(pos_hbm, send_hbm, recv_hbm, out_hbm,
                  idx_s, idx_r, rows_s, rows_r, out_v, sem):
    wid = lax.axis_index("s") * NC + lax.axis_index("c")
    lane = lax.iota(jnp.int32, L)
    col = [jnp.full((L,), k, jnp.int32) for k in range(4)]
    n_mine = (N_CHUNKS - wid + NW - 1) // NW

    def chunk_body(t, carry):
        g = wid + t * NW
        pltpu.sync_copy(send_hbm.at[g], idx_s)
        pltpu.sync_copy(recv_hbm.at[g], idx_r)
        copies = []
        for j in range(NSUB):
            dst_s = rows_s.at[pl.ds(j * SUB, SUB)]
            dst_r = rows_r.at[pl.ds(j * SUB, SUB)]
            copies.append(pltpu.async_copy(pos_hbm.at[idx_s.at[j]], dst_s, sem))
            copies.append(pltpu.async_copy(pos_hbm.at[idx_r.at[j]], dst_r, sem))
        for c in copies:
            c.wait()

        def step(i, carry2):
            row = i * L + lane
            d = []
            for k in range(3):
                sv = plsc.load_gather(rows_s, [row, col[k]])
                rv = plsc.load_gather(rows_r, [row, col[k]])
                d.append(rv - sv)
            acc = d[0] * d[0] + d[1] * d[1] + d[2] * d[2] + 1e-12
            ln = acc * _rsqrt(acc)
            for k in range(3):
                plsc.store_scatter(out_v, [row, col[k]], d[k])
            plsc.store_scatter(out_v, [row, col[3]], ln)
            return carry2

        lax.fori_loop(0, CHUNK // L, step, 0, unroll=2)
        pltpu.sync_copy(out_v, out_hbm.at[pl.ds(g * CHUNK, CHUNK)])
        return carry

    lax.fori_loop(0, n_mine, chunk_body, 0)


def kernel(positions, senders, receivers):
    send_t = senders.reshape(N_CHUNKS, NSUB, SUB)
    recv_t = receivers.reshape(N_CHUNKS, NSUB, SUB)
    return _edge_vectors(positions, send_t, recv_t)


# speedup vs baseline: 12.7395x; 1.0124x over previous
import functools

import jax
import jax.numpy as jnp
from jax import lax
from jax.experimental import pallas as pl
from jax.experimental.pallas import tpu as pltpu
from jax.experimental.pallas import tpu_sc as plsc

N_NODES = 100000
N_EDGES = 6400000

NC = 2
NS = 16
L = 16
NW = NC * NS

SUB = 128
NSUB = 16
CHUNK = SUB * NSUB
N_CHUNKS = N_EDGES // CHUNK

_MESH = plsc.VectorSubcoreMesh(core_axis_name="c", subcore_axis_name="s")


def _rsqrt(x):
    i = lax.bitcast_convert_type(x, jnp.int32)
    i = jnp.int32(0x5F3759DF) - lax.shift_right_logical(i, 1)
    y = lax.bitcast_convert_type(i, jnp.float32)
    for _ in range(2):
        y = y * (1.5 - 0.5 * x * y * y)
    return y


@functools.partial(
    pl.kernel,
    out_type=jax.ShapeDtypeStruct((N_EDGES, 4), jnp.float32),
    mesh=_MESH,
    scratch_types=[
        pltpu.VMEM((NSUB, SUB), jnp.int32),
        pltpu.VMEM((NSUB, SUB), jnp.int32),
        pltpu.VMEM((CHUNK, 3), jnp.float32),
        pltpu.VMEM((CHUNK, 3), jnp.float32),
        pltpu.VMEM((CHUNK, 4), jnp.float32),
        pltpu.SemaphoreType.DMA,
    ],
    compiler_params=pltpu.CompilerParams(
        needs_layout_passes=False, use_tc_tiling_on_sc=False),
)
def _edge_vectors(pos_hbm, send_hbm, recv_hbm, out_hbm,
                  idx_s, idx_r, rows_s, rows_r, out_v, sem):
    wid = lax.axis_index("s") * NC + lax.axis_index("c")
    lane = lax.iota(jnp.int32, L)
    col = [jnp.full((L,), k, jnp.int32) for k in range(4)]
    n_mine = (N_CHUNKS - wid + NW - 1) // NW

    def chunk_body(t, carry):
        g = wid + t * NW
        base = g * CHUNK
        pltpu.sync_copy(send_hbm.at[g], idx_s)
        pltpu.sync_copy(recv_hbm.at[g], idx_r)
        copies = []
        for j in range(NSUB):
            sl = pl.ds(j * SUB, SUB)
            copies.append(
                pltpu.async_copy(pos_hbm.at[idx_s.at[j]], rows_s.at[sl], sem))
            copies.append(
                pltpu.async_copy(pos_hbm.at[idx_r.at[j]], rows_r.at[sl], sem))
        for c in copies:
            c.wait()

        def step(i, carry2):
            row = i * L + lane
            d = []
            for k in range(3):
                sv = plsc.load_gather(rows_s, [row, col[k]])
                rv = plsc.load_gather(rows_r, [row, col[k]])
                d.append(rv - sv)
            acc = d[0] * d[0] + d[1] * d[1] + d[2] * d[2] + 1e-12
            ln = acc * _rsqrt(acc)
            for k in range(3):
                plsc.store_scatter(out_v, [row, col[k]], d[k])
            plsc.store_scatter(out_v, [row, col[3]], ln)
            return carry2

        lax.fori_loop(0, CHUNK // L, step, 0, unroll=4)
        pltpu.sync_copy(out_v, out_hbm.at[pl.ds(base, CHUNK)])
        return carry

    lax.fori_loop(0, n_mine, chunk_body, 0)


def kernel(positions, senders, receivers):
    send_t = senders.reshape(N_CHUNKS, NSUB, SUB)
    recv_t = receivers.reshape(N_CHUNKS, NSUB, SUB)
    return _edge_vectors(positions, send_t, recv_t)
